# baseline (speedup 1.0000x reference)
import jax
import jax.numpy as jnp
from jax import lax
from jax.experimental import pallas as pl
from jax.experimental.pallas import tpu as pltpu
from jax.experimental.pallas import tpu_sc as plsc

N = 10000
E = 320000
H = 64
STEPS = 8
OUT = 3

IDXR, IDXC = 2500, 128
GW = 4
EB = 4000
SN = N // 16
FPS = float(2 ** 23)

_mesh = plsc.VectorSubcoreMesh(core_axis_name="core", subcore_axis_name="subcore")
_sc_params = pltpu.CompilerParams(use_tc_tiling_on_sc=False)


def _sc_gather2(nf, snd2, rcv2):
    out_t = (jax.ShapeDtypeStruct((E, H), jnp.float32),
             jax.ShapeDtypeStruct((E, H), jnp.float32))

    @pl.kernel(out_type=out_t, mesh=_mesh, compiler_params=_sc_params,
               scratch_types=[pltpu.SemaphoreType.DMA])
    def k(t_hbm, si_hbm, ri_hbm, rs_hbm, rr_hbm, sem):
        def body(si, ri, rs, rr):
            cps = []
            for j in range(GW):
                cps.append(pltpu.async_copy(
                    t_hbm.at[si.at[j]], rs.at[pl.ds(j * IDXC, IDXC)], sem))
                cps.append(pltpu.async_copy(
                    t_hbm.at[ri.at[j]], rr.at[pl.ds(j * IDXC, IDXC)], sem))
            for c in cps:
                c.wait()

        pltpu.emit_pipeline(
            body,
            grid=(IDXR // GW,),
            in_specs=[pl.BlockSpec((GW, IDXC), lambda i: (i, 0)),
                      pl.BlockSpec((GW, IDXC), lambda i: (i, 0))],
            out_specs=[pl.BlockSpec((GW * IDXC, H), lambda i: (i, 0)),
                       pl.BlockSpec((GW * IDXC, H), lambda i: (i, 0))],
            core_axis_name=("core", "subcore"),
            dimension_semantics=(pltpu.PARALLEL,),
        )(si_hbm, ri_hbm, rs_hbm, rr_hbm)

    return k(nf, snd2, rcv2)


def _sc_scatter_add(e_upd_q, rcv2, zrows):
    @pl.kernel(out_type=jax.ShapeDtypeStruct((2, N, H), jnp.int32),
               mesh=_mesh, compiler_params=_sc_params,
               scratch_types=[pltpu.VMEM_SHARED((N, H), jnp.int32),
                              pltpu.SemaphoreType.DMA])
    def k(x_hbm, i_hbm, z_hbm, o_hbm, acc, sem):
        cid = lax.axis_index("core")
        sid = lax.axis_index("subcore")
        r0 = sid * SN
        pltpu.sync_copy(z_hbm, acc.at[pl.ds(r0, SN)])
        plsc.subcore_barrier()

        def body(x, i):
            cps = [pltpu.async_copy(x.at[pl.ds(j * IDXC, IDXC)],
                                    acc.at[i.at[j]], sem, add=True)
                   for j in range(GW)]
            for c in cps:
                c.wait()

        pltpu.emit_pipeline(
            body,
            grid=(IDXR // GW,),
            in_specs=[pl.BlockSpec((GW * IDXC, H), lambda i: (i, 0)),
                      pl.BlockSpec((GW, IDXC), lambda i: (i, 0))],
            out_specs=[],
            core_axis_name=("core", "subcore"),
            dimension_semantics=(pltpu.PARALLEL,),
        )(x_hbm, i_hbm)

        plsc.subcore_barrier()
        pltpu.sync_copy(acc.at[pl.ds(r0, SN)], o_hbm.at[cid, pl.ds(r0, SN)])

    return k(e_upd_q, rcv2, zrows)


def _ln(h, g, b):
    mu = jnp.mean(h, axis=-1, keepdims=True)
    var = jnp.mean((h - mu) ** 2, axis=-1, keepdims=True)
    return (h - mu) / jnp.sqrt(var + 1e-5) * g + b


def _dot(a, b):
    return jnp.dot(a, b, preferred_element_type=jnp.float32)


def _node_encoder(x, nW1, nb1, nW2, nb2, nlg, nlb):
    def body(x_r, w1, b1, w2, b2, g, b, nfo):
        h = jnp.maximum(_dot(x_r[...], w1[...]) + b1[...], 0.0)
        nfo[...] = _ln(_dot(h, w2[...]) + b2[...], g[...], b[...])

    return pl.pallas_call(
        body,
        out_shape=jax.ShapeDtypeStruct((N, H), jnp.float32),
    )(x, nW1, nb1, nW2, nb2, nlg, nlb)


def _edge_encoder(edge_attr, eW1, eb1, eW2, eb2, elg, elb):
    def body(a_r, w1, b1, w2, b2, g, b, efo):
        h = jnp.maximum(_dot(a_r[...], w1[...]) + b1[...], 0.0)
        efo[...] = _ln(_dot(h, w2[...]) + b2[...], g[...], b[...])

    D = edge_attr.shape[1]
    wspec = lambda a, b: pl.BlockSpec((a, b), lambda i: (0, 0))
    return pl.pallas_call(
        body,
        grid=(E // EB,),
        in_specs=[pl.BlockSpec((EB, D), lambda i: (i, 0)),
                  wspec(D, H), wspec(1, H), wspec(H, H), wspec(1, H),
                  wspec(1, H), wspec(1, H)],
        out_specs=pl.BlockSpec((EB, H), lambda i: (i, 0)),
        out_shape=jax.ShapeDtypeStruct((E, H), jnp.float32),
    )(edge_attr, eW1, eb1, eW2, eb2, elg, elb)


def _edge_mlp(Rs, Rr, ef, W1, b1, W2, b2, g, b):
    def body(rs, rr, ef_r, w1, b1r, w2, b2r, gr, br, eu, efn):
        e_in = jnp.concatenate([rs[...], rr[...], ef_r[...]], axis=1)
        h = jnp.maximum(_dot(e_in, w1[...]) + b1r[...], 0.0)
        e = _ln(_dot(h, w2[...]) + b2r[...], gr[...], br[...])
        eu[...] = jnp.floor(e * FPS + 0.5).astype(jnp.int32)
        efn[...] = ef_r[...] + e

    blk = lambda: pl.BlockSpec((EB, H), lambda i: (i, 0))
    wspec = lambda a, b: pl.BlockSpec((a, b), lambda i: (0, 0))
    return pl.pallas_call(
        body,
        grid=(E // EB,),
        in_specs=[blk(), blk(), blk(), wspec(3 * H, H), wspec(1, H),
                  wspec(H, H), wspec(1, H), wspec(1, H), wspec(1, H)],
        out_specs=[blk(), blk()],
        out_shape=[jax.ShapeDtypeStruct((E, H), jnp.int32),
                   jax.ShapeDtypeStruct((E, H), jnp.float32)],
    )(Rs, Rr, ef, W1, b1, W2, b2, g, b)


def _node_mlp(nf, aggs, W1, b1, W2, b2, g, b):
    def body(nf_r, ag, w1, b1r, w2, b2r, gr, br, nfo):
        agg = (ag[0] + ag[1]).astype(jnp.float32) * (1.0 / FPS)
        n_in = jnp.concatenate([nf_r[...], agg], axis=1)
        h = jnp.maximum(_dot(n_in, w1[...]) + b1r[...], 0.0)
        nu = _ln(_dot(h, w2[...]) + b2r[...], gr[...], br[...])
        nfo[...] = nf_r[...] + nu

    return pl.pallas_call(
        body,
        out_shape=jax.ShapeDtypeStruct((N, H), jnp.float32),
    )(nf, aggs, W1, b1, W2, b2, g, b)


def _node_mlp_last(nf, aggs, W1, b1, W2, b2, g, b, dW1, db1, dW2, db2):
    def body(nf_r, ag, w1, b1r, w2, b2r, gr, br, w1d, b1d, w2d, b2d, o):
        agg = (ag[0] + ag[1]).astype(jnp.float32) * (1.0 / FPS)
        n_in = jnp.concatenate([nf_r[...], agg], axis=1)
        h = jnp.maximum(_dot(n_in, w1[...]) + b1r[...], 0.0)
        nu = _ln(_dot(h, w2[...]) + b2r[...], gr[...], br[...])
        nfn = nf_r[...] + nu
        hd = jnp.maximum(_dot(nfn, w1d[...]) + b1d[...], 0.0)
        o[...] = _dot(hd, w2d[...]) + b2d[...]

    return pl.pallas_call(
        body,
        out_shape=jax.ShapeDtypeStruct((N, OUT), jnp.float32),
    )(nf, aggs, W1, b1, W2, b2, g, b, dW1, db1, dW2, db2)


def kernel(x, edge_attr, edge_index, nW1, nb1, nW2, nb2, nlg, nlb,
           eW1, eb1, eW2, eb2, elg, elb,
           beW1, beb1, beW2, beb2, belg, belb,
           bnW1, bnb1, bnW2, bnb2, bnlg, bnlb,
           dW1, db1, dW2, db2):
    r1 = lambda v: v.reshape(1, -1)
    snd2 = edge_index[0].reshape(IDXR, IDXC)
    rcv2 = edge_index[1].reshape(IDXR, IDXC)
    zrows = jnp.zeros((SN, H), jnp.int32)

    nf = _node_encoder(x, nW1, r1(nb1), nW2, r1(nb2), r1(nlg), r1(nlb))
    ef = _edge_encoder(edge_attr, eW1, r1(eb1), eW2, r1(eb2), r1(elg), r1(elb))

    for i in range(STEPS):
        Rs, Rr = _sc_gather2(nf, snd2, rcv2)
        e_upd, ef = _edge_mlp(Rs, Rr, ef, beW1[i], r1(beb1[i]),
                              beW2[i], r1(beb2[i]), r1(belg[i]), r1(belb[i]))
        aggs = _sc_scatter_add(e_upd, rcv2, zrows)
        if i < STEPS - 1:
            nf = _node_mlp(nf, aggs, bnW1[i], r1(bnb1[i]), bnW2[i],
                           r1(bnb2[i]), r1(bnlg[i]), r1(bnlb[i]))
        else:
            out = _node_mlp_last(nf, aggs, bnW1[i], r1(bnb1[i]), bnW2[i],
                                 r1(bnb2[i]), r1(bnlg[i]), r1(bnlb[i]),
                                 dW1, r1(db1), dW2, r1(db2))
    return out


# speedup vs baseline: 1.0142x; 1.0142x over previous
import jax
import jax.numpy as jnp
from jax import lax
from jax.experimental import pallas as pl
from jax.experimental.pallas import tpu as pltpu
from jax.experimental.pallas import tpu_sc as plsc

N = 10000
E = 320000
H = 64
STEPS = 8
OUT = 3

IDXR, IDXC = 2500, 128
NHALF = 2
HIDXR = IDXR // NHALF
EH = E // NHALF
GWG = 2
GWS = 5
EB = 4000
SN = N // 16
FPS = float(2 ** 23)

_mesh = plsc.VectorSubcoreMesh(core_axis_name="core", subcore_axis_name="subcore")
_sc_params = pltpu.CompilerParams(use_tc_tiling_on_sc=False)


def _sc_gather2(nf, snd2, rcv2, off):
    out_t = (jax.ShapeDtypeStruct((EH, H), jnp.float32),
             jax.ShapeDtypeStruct((EH, H), jnp.float32))

    @pl.kernel(out_type=out_t, mesh=_mesh, compiler_params=_sc_params,
               scratch_types=[pltpu.SemaphoreType.DMA])
    def k(t_hbm, si_hbm, ri_hbm, rs_hbm, rr_hbm, sem):
        def body(si, ri, rs, rr):
            cps = []
            for j in range(GWG):
                cps.append(pltpu.async_copy(
                    t_hbm.at[si.at[j]], rs.at[pl.ds(j * IDXC, IDXC)], sem))
                cps.append(pltpu.async_copy(
                    t_hbm.at[ri.at[j]], rr.at[pl.ds(j * IDXC, IDXC)], sem))
            for c in cps:
                c.wait()

        pltpu.emit_pipeline(
            body,
            grid=(HIDXR // GWG,),
            in_specs=[pl.BlockSpec((GWG, IDXC), lambda i, o=off: (i + o, 0)),
                      pl.BlockSpec((GWG, IDXC), lambda i, o=off: (i + o, 0))],
            out_specs=[pl.BlockSpec((GWG * IDXC, H), lambda i: (i, 0)),
                       pl.BlockSpec((GWG * IDXC, H), lambda i: (i, 0))],
            core_axis_name=("core", "subcore"),
            dimension_semantics=(pltpu.PARALLEL,),
        )(si_hbm, ri_hbm, rs_hbm, rr_hbm)

    return k(nf, snd2, rcv2)


def _sc_scatter_add(e_upd_q, rcv2, zrows, off):
    @pl.kernel(out_type=jax.ShapeDtypeStruct((2, N, H), jnp.int32),
               mesh=_mesh, compiler_params=_sc_params,
               scratch_types=[pltpu.VMEM_SHARED((N, H), jnp.int32),
                              pltpu.SemaphoreType.DMA])
    def k(x_hbm, i_hbm, z_hbm, o_hbm, acc, sem):
        cid = lax.axis_index("core")
        sid = lax.axis_index("subcore")
        r0 = sid * SN
        pltpu.sync_copy(z_hbm, acc.at[pl.ds(r0, SN)])
        plsc.subcore_barrier()

        def body(x, i):
            cps = [pltpu.async_copy(x.at[pl.ds(j * IDXC, IDXC)],
                                    acc.at[i.at[j]], sem, add=True)
                   for j in range(GWS)]
            for c in cps:
                c.wait()

        pltpu.emit_pipeline(
            body,
            grid=(HIDXR // GWS,),
            in_specs=[pl.BlockSpec((GWS * IDXC, H), lambda i: (i, 0)),
                      pl.BlockSpec((GWS, IDXC), lambda i, o=off: (i + o, 0))],
            out_specs=[],
            core_axis_name=("core", "subcore"),
            dimension_semantics=(pltpu.PARALLEL,),
        )(x_hbm, i_hbm)

        plsc.subcore_barrier()
        pltpu.sync_copy(acc.at[pl.ds(r0, SN)], o_hbm.at[cid, pl.ds(r0, SN)])

    return k(e_upd_q, rcv2, zrows)


def _ln(h, g, b):
    mu = jnp.mean(h, axis=-1, keepdims=True)
    var = jnp.mean((h - mu) ** 2, axis=-1, keepdims=True)
    return (h - mu) / jnp.sqrt(var + 1e-5) * g + b


def _dot(a, b):
    return jnp.dot(a, b, preferred_element_type=jnp.float32)


def _node_encoder(x, nW1, nb1, nW2, nb2, nlg, nlb):
    def body(x_r, w1, b1, w2, b2, g, b, nfo):
        h = jnp.maximum(_dot(x_r[...], w1[...]) + b1[...], 0.0)
        nfo[...] = _ln(_dot(h, w2[...]) + b2[...], g[...], b[...])

    return pl.pallas_call(
        body,
        out_shape=jax.ShapeDtypeStruct((N, H), jnp.float32),
    )(x, nW1, nb1, nW2, nb2, nlg, nlb)


def _edge_encoder(edge_attr, eW1, eb1, eW2, eb2, elg, elb, boff):
    def body(a_r, w1, b1, w2, b2, g, b, efo):
        h = jnp.maximum(_dot(a_r[...], w1[...]) + b1[...], 0.0)
        efo[...] = _ln(_dot(h, w2[...]) + b2[...], g[...], b[...])

    D = edge_attr.shape[1]
    wspec = lambda a, b: pl.BlockSpec((a, b), lambda i: (0, 0))
    return pl.pallas_call(
        body,
        grid=(EH // EB,),
        in_specs=[pl.BlockSpec((EB, D), lambda i, o=boff: (i + o, 0)),
                  wspec(D, H), wspec(1, H), wspec(H, H), wspec(1, H),
                  wspec(1, H), wspec(1, H)],
        out_specs=pl.BlockSpec((EB, H), lambda i: (i, 0)),
        out_shape=jax.ShapeDtypeStruct((EH, H), jnp.float32),
    )(edge_attr, eW1, eb1, eW2, eb2, elg, elb)


def _edge_mlp(Rs, Rr, ef, W1, b1, W2, b2, g, b):
    def body(rs, rr, ef_r, w1, b1r, w2, b2r, gr, br, eu, efn):
        e_in = jnp.concatenate([rs[...], rr[...], ef_r[...]], axis=1)
        h = jnp.maximum(_dot(e_in, w1[...]) + b1r[...], 0.0)
        e = _ln(_dot(h, w2[...]) + b2r[...], gr[...], br[...])
        eu[...] = jnp.floor(e * FPS + 0.5).astype(jnp.int32)
        efn[...] = ef_r[...] + e

    blk = lambda: pl.BlockSpec((EB, H), lambda i: (i, 0))
    wspec = lambda a, b: pl.BlockSpec((a, b), lambda i: (0, 0))
    return pl.pallas_call(
        body,
        grid=(EH // EB,),
        in_specs=[blk(), blk(), blk(), wspec(3 * H, H), wspec(1, H),
                  wspec(H, H), wspec(1, H), wspec(1, H), wspec(1, H)],
        out_specs=[blk(), blk()],
        out_shape=[jax.ShapeDtypeStruct((EH, H), jnp.int32),
                   jax.ShapeDtypeStruct((EH, H), jnp.float32)],
    )(Rs, Rr, ef, W1, b1, W2, b2, g, b)


def _merge_agg(agA, agB):
    return (agA[0] + agA[1] + agB[0] + agB[1]).astype(jnp.float32) * (1.0 / FPS)


def _node_mlp(nf, aggA, aggB, W1, b1, W2, b2, g, b):
    def body(nf_r, agA, agB, w1, b1r, w2, b2r, gr, br, nfo):
        n_in = jnp.concatenate([nf_r[...], _merge_agg(agA[...], agB[...])],
                               axis=1)
        h = jnp.maximum(_dot(n_in, w1[...]) + b1r[...], 0.0)
        nu = _ln(_dot(h, w2[...]) + b2r[...], gr[...], br[...])
        nfo[...] = nf_r[...] + nu

    return pl.pallas_call(
        body,
        out_shape=jax.ShapeDtypeStruct((N, H), jnp.float32),
    )(nf, aggA, aggB, W1, b1, W2, b2, g, b)


def _node_mlp_last(nf, aggA, aggB, W1, b1, W2, b2, g, b, dW1, db1, dW2, db2):
    def body(nf_r, agA, agB, w1, b1r, w2, b2r, gr, br, w1d, b1d, w2d, b2d, o):
        n_in = jnp.concatenate([nf_r[...], _merge_agg(agA[...], agB[...])],
                               axis=1)
        h = jnp.maximum(_dot(n_in, w1[...]) + b1r[...], 0.0)
        nu = _ln(_dot(h, w2[...]) + b2r[...], gr[...], br[...])
        nfn = nf_r[...] + nu
        hd = jnp.maximum(_dot(nfn, w1d[...]) + b1d[...], 0.0)
        o[...] = _dot(hd, w2d[...]) + b2d[...]

    return pl.pallas_call(
        body,
        out_shape=jax.ShapeDtypeStruct((N, OUT), jnp.float32),
    )(nf, aggA, aggB, W1, b1, W2, b2, g, b, dW1, db1, dW2, db2)


def kernel(x, edge_attr, edge_index, nW1, nb1, nW2, nb2, nlg, nlb,
           eW1, eb1, eW2, eb2, elg, elb,
           beW1, beb1, beW2, beb2, belg, belb,
           bnW1, bnb1, bnW2, bnb2, bnlg, bnlb,
           dW1, db1, dW2, db2):
    r1 = lambda v: v.reshape(1, -1)
    snd2 = edge_index[0].reshape(IDXR, IDXC)
    rcv2 = edge_index[1].reshape(IDXR, IDXC)
    zrows = jnp.zeros((SN, H), jnp.int32)
    goffs = (0, HIDXR // GWG)
    soffs = (0, HIDXR // GWS)
    boffs = (0, EH // EB)

    nf = _node_encoder(x, nW1, r1(nb1), nW2, r1(nb2), r1(nlg), r1(nlb))
    efs = [_edge_encoder(edge_attr, eW1, r1(eb1), eW2, r1(eb2), r1(elg),
                         r1(elb), boffs[hh]) for hh in range(NHALF)]

    for i in range(STEPS):
        b1i, b2i = r1(beb1[i]), r1(beb2[i])
        lgi, lbi = r1(belg[i]), r1(belb[i])
        gs = [_sc_gather2(nf, snd2, rcv2, goffs[hh]) for hh in range(NHALF)]
        eqs = []
        for hh in range(NHALF):
            eq, efn = _edge_mlp(gs[hh][0], gs[hh][1], efs[hh], beW1[i], b1i,
                                beW2[i], b2i, lgi, lbi)
            eqs.append(eq)
            efs[hh] = efn
        aggs = [_sc_scatter_add(eqs[hh], rcv2, zrows, soffs[hh])
                for hh in range(NHALF)]
        if i < STEPS - 1:
            nf = _node_mlp(nf, aggs[0], aggs[1], bnW1[i], r1(bnb1[i]),
                           bnW2[i], r1(bnb2[i]), r1(bnlg[i]), r1(bnlb[i]))
        else:
            out = _node_mlp_last(nf, aggs[0], aggs[1], bnW1[i], r1(bnb1[i]),
                                 bnW2[i], r1(bnb2[i]), r1(bnlg[i]),
                                 r1(bnlb[i]), dW1, r1(db1), dW2, r1(db2))
    return out
